# baseline (speedup 1.0000x reference)
import jax
import jax.numpy as jnp
from jax import lax
from jax.experimental import pallas as pl
from jax.experimental.pallas import tpu as pltpu
from jax.experimental.pallas import tpu_sc as plsc

C = 16
D = 32
V = 1000000
N = 16384
I = 4096

G = 4
NC = 2
NS = 16
NW = NC * NS
ROWS_PER_W = N // NW
KCH = ROWS_PER_W // 128
STRIPE = I // NS


def _gather_sc(table4_hbm, idx_hbm, out_hbm, idx_v, gidx_v, rows_v, sem):
    wid = lax.axis_index("s") * NC + lax.axis_index("c")
    base = wid * ROWS_PER_W
    pltpu.sync_copy(idx_hbm.at[wid], idx_v)
    for j in range(KCH):
        for k in range(8):
            sl = pl.ds(k * 16, 16)
            gidx_v[j, sl] = lax.shift_right_logical(idx_v[j, sl], 2)
    handles = [
        pltpu.async_copy(table4_hbm.at[gidx_v.at[j]],
                         rows_v.at[pl.ds(j * 128, 128)], sem)
        for j in range(KCH)
    ]
    for h in handles:
        h.wait()
    pltpu.sync_copy(rows_v, out_hbm.at[pl.ds(base, ROWS_PER_W)])


def _ll_tc(rows4_ref, mu_ref, anno_ref, ann_ref, conf_ref, out_ref):
    rows4 = rows4_ref[...]
    ann = ann_ref[...]
    sel = jnp.bitwise_and(ann, G - 1)
    rows = jnp.zeros((rows4.shape[0], D), jnp.float32)
    for k in range(G):
        pick = (sel == k).astype(jnp.float32)
        rows = rows + rows4[:, k * D:(k + 1) * D] * pick
    e_mu = jnp.exp(mu_ref[...])
    ee = jnp.exp(e_mu)
    er = jnp.exp(rows)
    s = lax.dot_general(er, ee, (((1,), (1,)), ((), ())),
                        preferred_element_type=jnp.float32)
    anno = anno_ref[...]
    onehot = (anno == lax.broadcasted_iota(jnp.int32, rows.shape, 1)
              ).astype(jnp.float32)
    r_an = jnp.sum(rows * onehot, axis=1, keepdims=True)
    e_an = lax.dot_general(onehot, e_mu, (((1,), (1,)), ((), ())),
                           preferred_element_type=jnp.float32)
    out_ref[...] = (e_an + r_an - jnp.log(s)) * conf_ref[...]


def _scatter_sc(ll_hbm, items_hbm, out_hbm, idx_v, ll_v, zbuf, acc_sh, sem):
    del sem
    cid = lax.axis_index("c")
    sid = lax.axis_index("s")
    wid = sid * NC + cid
    base = wid * ROWS_PER_W
    pltpu.sync_copy(items_hbm.at[wid], idx_v)
    pltpu.sync_copy(ll_hbm.at[pl.ds(base, ROWS_PER_W)], ll_v)

    def _zero_row(j, carry):
        zbuf[j, :] = jnp.zeros((C,), jnp.float32)
        return carry
    lax.fori_loop(0, STRIPE, _zero_row, 0)
    pltpu.sync_copy(zbuf, acc_sh.at[pl.ds(sid * STRIPE, STRIPE)])
    plsc.subcore_barrier()
    for j in range(KCH):
        pltpu.sync_copy(ll_v.at[pl.ds(j * 128, 128)],
                        acc_sh.at[idx_v.at[j]], add=True)
    plsc.subcore_barrier()
    pltpu.sync_copy(acc_sh.at[pl.ds(sid * STRIPE, STRIPE)], zbuf)
    pltpu.sync_copy(zbuf, out_hbm.at[cid, pl.ds(sid * STRIPE, STRIPE)])


def _combine_tc(parts_ref, out_ref):
    out_ref[...] = (parts_ref[0] + parts_ref[1]).T


def kernel(mu, random_effects, anno, items, annotators, confidences):
    mesh = plsc.VectorSubcoreMesh(core_axis_name="c", subcore_axis_name="s")

    gather = pl.kernel(
        _gather_sc, mesh=mesh,
        out_type=jax.ShapeDtypeStruct((N, G * D), jnp.float32),
        scratch_types=[
            pltpu.VMEM((KCH, 128), jnp.int32),
            pltpu.VMEM((KCH, 128), jnp.int32),
            pltpu.VMEM((ROWS_PER_W, G * D), jnp.float32),
            pltpu.SemaphoreType.DMA,
        ],
    )
    ann32 = annotators.astype(jnp.int32)
    rows4 = gather(random_effects.reshape(V // G, G * D),
                   ann32.reshape(NW, KCH, 128))

    grid = 8
    blk = N // grid
    ll = pl.pallas_call(
        _ll_tc,
        grid=(grid,),
        in_specs=[
            pl.BlockSpec((blk, G * D), lambda i: (i, 0)),
            pl.BlockSpec((C, D), lambda i: (0, 0)),
            pl.BlockSpec((blk, 1), lambda i: (i, 0)),
            pl.BlockSpec((blk, 1), lambda i: (i, 0)),
            pl.BlockSpec((blk, 1), lambda i: (i, 0)),
        ],
        out_specs=pl.BlockSpec((blk, C), lambda i: (i, 0)),
        out_shape=jax.ShapeDtypeStruct((N, C), jnp.float32),
    )(rows4, mu, anno.astype(jnp.int32).reshape(N, 1),
      ann32.reshape(N, 1), confidences.reshape(N, 1))

    scatter = pl.kernel(
        _scatter_sc, mesh=mesh,
        compiler_params=pltpu.CompilerParams(use_tc_tiling_on_sc=False),
        out_type=jax.ShapeDtypeStruct((NC, I, C), jnp.float32),
        scratch_types=[
            pltpu.VMEM((KCH, 128), jnp.int32),
            pltpu.VMEM((ROWS_PER_W, C), jnp.float32),
            pltpu.VMEM((STRIPE, C), jnp.float32),
            pltpu.VMEM_SHARED((I, C), jnp.float32),
            pltpu.SemaphoreType.DMA,
        ],
    )
    parts = scatter(ll, items.astype(jnp.int32).reshape(NW, KCH, 128))

    return pl.pallas_call(
        _combine_tc,
        out_shape=jax.ShapeDtypeStruct((C, I), jnp.float32),
    )(parts)


# speedup vs baseline: 1.5221x; 1.5221x over previous
import jax
import jax.numpy as jnp
from jax import lax
from jax.experimental import pallas as pl
from jax.experimental.pallas import tpu as pltpu
from jax.experimental.pallas import tpu_sc as plsc

C = 16
D = 32
V = 1000000
N = 16384
I = 4096

G = 4
NC = 2
NS = 16
NW = NC * NS
ROWS_PER_W = N // NW
KCH = ROWS_PER_W // 128
STRIPE = I // NS


TBL = 1024
NBLK = 245
R = NBLK * TBL


def _repack_tc(t0_ref, t1_ref, t2_ref, t3_ref, out_ref):
    parts = [t0_ref[...].T, t1_ref[...].T, t2_ref[...].T, t3_ref[...].T]
    out_ref[...] = jnp.concatenate(parts, axis=1)


def _gather_sc(table4_hbm, idx_hbm, out_hbm, idx_v, rows_v, sem):
    wid = lax.axis_index("s") * NC + lax.axis_index("c")
    base = wid * ROWS_PER_W
    pltpu.sync_copy(idx_hbm.at[wid], idx_v)
    handles = [
        pltpu.async_copy(table4_hbm.at[idx_v.at[j]],
                         rows_v.at[pl.ds(j * 128, 128)], sem)
        for j in range(KCH)
    ]
    for h in handles:
        h.wait()
    pltpu.sync_copy(rows_v, out_hbm.at[pl.ds(base, ROWS_PER_W)])


def _ll_tc(rows4_ref, mu_ref, anno_ref, ann_ref, conf_ref, out_ref):
    rows4 = rows4_ref[...]
    ann = ann_ref[...]
    sel = ((ann >= R).astype(jnp.int32) + (ann >= 2 * R).astype(jnp.int32)
           + (ann >= 3 * R).astype(jnp.int32))
    rows = jnp.zeros((rows4.shape[0], D), jnp.float32)
    for k in range(G):
        pick = (sel == k).astype(jnp.float32)
        rows = rows + rows4[:, k * D:(k + 1) * D] * pick
    e_mu = jnp.exp(mu_ref[...])
    ee = jnp.exp(e_mu)
    er = jnp.exp(rows)
    s = lax.dot_general(er, ee, (((1,), (1,)), ((), ())),
                        preferred_element_type=jnp.float32)
    anno = anno_ref[...]
    onehot = (anno == lax.broadcasted_iota(jnp.int32, rows.shape, 1)
              ).astype(jnp.float32)
    r_an = jnp.sum(rows * onehot, axis=1, keepdims=True)
    e_an = lax.dot_general(onehot, e_mu, (((1,), (1,)), ((), ())),
                           preferred_element_type=jnp.float32)
    out_ref[...] = (e_an + r_an - jnp.log(s)) * conf_ref[...]


def _scatter_sc(ll_hbm, items_hbm, out_hbm, idx_v, ll_v, zbuf, acc_sh, sem):
    del sem
    cid = lax.axis_index("c")
    sid = lax.axis_index("s")
    wid = sid * NC + cid
    base = wid * ROWS_PER_W
    pltpu.sync_copy(items_hbm.at[wid], idx_v)
    pltpu.sync_copy(ll_hbm.at[pl.ds(base, ROWS_PER_W)], ll_v)

    def _zero_row(j, carry):
        zbuf[j, :] = jnp.zeros((C,), jnp.float32)
        return carry
    lax.fori_loop(0, STRIPE, _zero_row, 0)
    pltpu.sync_copy(zbuf, acc_sh.at[pl.ds(sid * STRIPE, STRIPE)])
    plsc.subcore_barrier()
    for j in range(KCH):
        pltpu.sync_copy(ll_v.at[pl.ds(j * 128, 128)],
                        acc_sh.at[idx_v.at[j]], add=True)
    plsc.subcore_barrier()
    pltpu.sync_copy(acc_sh.at[pl.ds(sid * STRIPE, STRIPE)], zbuf)
    pltpu.sync_copy(zbuf, out_hbm.at[cid, pl.ds(sid * STRIPE, STRIPE)])


def _combine_tc(parts_ref, out_ref):
    out_ref[...] = (parts_ref[0] + parts_ref[1]).T


def kernel(mu, random_effects, anno, items, annotators, confidences):
    mesh = plsc.VectorSubcoreMesh(core_axis_name="c", subcore_axis_name="s")

    gather = pl.kernel(
        _gather_sc, mesh=mesh,
        out_type=jax.ShapeDtypeStruct((N, G * D), jnp.float32),
        scratch_types=[
            pltpu.VMEM((KCH, 128), jnp.int32),
            pltpu.VMEM((ROWS_PER_W, G * D), jnp.float32),
            pltpu.SemaphoreType.DMA,
        ],
    )
    tT = random_effects.T
    table4 = pl.pallas_call(
        _repack_tc,
        grid=(NBLK,),
        in_specs=[
            pl.BlockSpec((D, TBL),
                         lambda i, c=c: (0, jnp.minimum(NBLK * c + i,
                                                        (V - 1) // TBL)))
            for c in range(G)
        ],
        out_specs=pl.BlockSpec((TBL, G * D), lambda i: (i, 0)),
        out_shape=jax.ShapeDtypeStruct((R, G * D), jnp.float32),
    )(tT, tT, tT, tT)

    ann32 = annotators.astype(jnp.int32)
    rows4 = gather(table4, (ann32 % R).reshape(NW, KCH, 128))

    grid = 8
    blk = N // grid
    ll = pl.pallas_call(
        _ll_tc,
        grid=(grid,),
        in_specs=[
            pl.BlockSpec((blk, G * D), lambda i: (i, 0)),
            pl.BlockSpec((C, D), lambda i: (0, 0)),
            pl.BlockSpec((blk, 1), lambda i: (i, 0)),
            pl.BlockSpec((blk, 1), lambda i: (i, 0)),
            pl.BlockSpec((blk, 1), lambda i: (i, 0)),
        ],
        out_specs=pl.BlockSpec((blk, C), lambda i: (i, 0)),
        out_shape=jax.ShapeDtypeStruct((N, C), jnp.float32),
    )(rows4, mu, anno.astype(jnp.int32).reshape(N, 1),
      ann32.reshape(N, 1), confidences.reshape(N, 1))

    scatter = pl.kernel(
        _scatter_sc, mesh=mesh,
        compiler_params=pltpu.CompilerParams(use_tc_tiling_on_sc=False),
        out_type=jax.ShapeDtypeStruct((NC, I, C), jnp.float32),
        scratch_types=[
            pltpu.VMEM((KCH, 128), jnp.int32),
            pltpu.VMEM((ROWS_PER_W, C), jnp.float32),
            pltpu.VMEM((STRIPE, C), jnp.float32),
            pltpu.VMEM_SHARED((I, C), jnp.float32),
            pltpu.SemaphoreType.DMA,
        ],
    )
    parts = scatter(ll, items.astype(jnp.int32).reshape(NW, KCH, 128))

    return pl.pallas_call(
        _combine_tc,
        out_shape=jax.ShapeDtypeStruct((C, I), jnp.float32),
    )(parts)
